# baseline (speedup 1.0000x reference)
import functools

import jax
import jax.numpy as jnp
from jax import lax
from jax.experimental import pallas as pl
from jax.experimental.pallas import tpu as pltpu
from jax.experimental.pallas import tpu_sc as plsc

_LANES = 16
_CH = 16


@jax.jit
def _gather(tbl, idx0, idx1):
    info = plsc.get_sparse_core_info()
    nc, ns = info.num_cores, info.num_subcores
    nw = nc * ns
    batch = idx0.shape[0]
    d = 32
    b_per_w = batch // nw
    n_chunks = b_per_w // _CH

    idx0_r = idx0.reshape(nw, b_per_w)
    idx1_r = idx1.reshape(nw, b_per_w)

    mesh = plsc.VectorSubcoreMesh(core_axis_name="c", subcore_axis_name="s")

    @functools.partial(
        pl.kernel,
        mesh=mesh,
        out_type=jax.ShapeDtypeStruct((batch, d), jnp.float32),
        compiler_params=pltpu.CompilerParams(needs_layout_passes=False),
        scratch_types=[
            pltpu.VMEM((b_per_w,), jnp.int32),
            pltpu.VMEM((b_per_w,), jnp.int32),
            pltpu.VMEM((_CH, d, 128), jnp.float32),
            pltpu.VMEM((_CH, d), jnp.float32),
            pltpu.SemaphoreType.DMA,
            pltpu.SemaphoreType.DMA,
        ],
    )
    def k(tbl_hbm, idx0_hbm, idx1_hbm, out_hbm,
          i0_v, i1_v, staged, outbuf, sem_g, sem_o):
        wid = lax.axis_index("s") * nc + lax.axis_index("c")
        pltpu.sync_copy(idx0_hbm.at[wid], i0_v)
        pltpu.sync_copy(idx1_hbm.at[wid], i1_v)

        def chunk_body(ci, _):
            base = ci * _CH
            t_vec = i0_v[pl.ds(base, _CH)]
            r_vec = i1_v[pl.ds(base, _CH)]
            copies = []
            for j in range(_CH):
                t = t_vec[j]
                r = r_vec[j]
                col = pl.multiple_of((r >> 7) << 7, 128)
                row0 = pl.multiple_of(t * d, d)
                copies.append(
                    pltpu.async_copy(
                        tbl_hbm.at[pl.ds(row0, d), pl.ds(col, 128)],
                        staged.at[j],
                        sem_g,
                    )
                )
            for c in copies:
                c.wait()
            lane_vec = r_vec & 127
            for j in range(_CH):
                lane = lane_vec[j]
                lane_v = jnp.full((_LANES,), lane, jnp.int32)
                j_v = jnp.full((_LANES,), j, jnp.int32)
                for h in range(d // _LANES):
                    c_v = lax.iota(jnp.int32, _LANES) + h * _LANES
                    vals = plsc.load_gather(staged, [j_v, c_v, lane_v])
                    outbuf[j, pl.ds(h * _LANES, _LANES)] = vals
            out_row = pl.multiple_of(wid * b_per_w + base, _CH)
            pltpu.async_copy(
                outbuf, out_hbm.at[pl.ds(out_row, _CH)], sem_o
            ).wait()
            return _

        lax.fori_loop(0, n_chunks, chunk_body, None)

    return k(tbl, idx0_r, idx1_r)


def kernel(train_table, indices):
    n_tables, vocab, d = train_table.shape
    tbl = jnp.transpose(train_table, (0, 2, 1)).reshape(n_tables * d, vocab)
    idx0 = indices[:, 0].astype(jnp.int32)
    idx1 = indices[:, 1].astype(jnp.int32)
    return _gather(tbl, idx0, idx1)
